# baseline (speedup 1.0000x reference)
import functools

import jax
import jax.numpy as jnp
from jax import lax
from jax.experimental import pallas as pl
from jax.experimental.pallas import tpu as pltpu
from jax.experimental.pallas import tpu_sc as plsc

D_MODEL = 64
NC = 2
NS = 16
NW = NC * NS
CHUNK = 128
K = 2
GROUP = CHUNK * K


@functools.lru_cache(maxsize=None)
def _build(n_groups, v_half):
    mesh = plsc.VectorSubcoreMesh(core_axis_name="c", subcore_axis_name="s")
    n_chunks = n_groups * K
    rows_per_w = n_groups * CHUNK
    out_rows = NW * rows_per_w

    @functools.partial(
        pl.kernel,
        mesh=mesh,
        out_type=jax.ShapeDtypeStruct((out_rows, 2 * D_MODEL), jnp.float32),
        scratch_types=[
            pltpu.VMEM((n_chunks, CHUNK), jnp.int32),
            pltpu.VMEM((GROUP, 2 * D_MODEL), jnp.float32),
            pltpu.VMEM((GROUP, 2 * D_MODEL), jnp.float32),
            pltpu.SemaphoreType.DMA,
            pltpu.SemaphoreType.DMA,
        ],
    )
    def body(table_hbm, idx_hbm, out_hbm, idx_v, rows0, rows1, sem0, sem1):
        wid = lax.axis_index("s") * NC + lax.axis_index("c")
        pltpu.sync_copy(idx_hbm.at[wid], idx_v)
        base = wid * rows_per_w

        def fire(g, buf, sem):
            for k in range(K):
                pltpu.async_copy(
                    table_hbm.at[idx_v.at[g * K + k]],
                    buf.at[pl.ds(k * CHUNK, CHUNK)],
                    sem,
                )

        def drain_writeback(g, buf, sem):
            pltpu.make_async_copy(table_hbm.at[pl.ds(0, GROUP)], buf, sem).wait()
            off = jnp.minimum(base + g * CHUNK, out_rows - GROUP)
            pltpu.sync_copy(buf, out_hbm.at[pl.ds(off, GROUP)])

        fire(0, rows0, sem0)

        def step(go, carry):
            g = 2 * go
            fire(g + 1, rows1, sem1)
            drain_writeback(g, rows0, sem0)
            fire(g + 2, rows0, sem0)
            drain_writeback(g + 1, rows1, sem1)
            return carry

        lax.fori_loop(0, n_groups // 2 - 1, step, 0)

        g_last = n_groups - 2
        fire(g_last + 1, rows1, sem1)
        drain_writeback(g_last, rows0, sem0)
        drain_writeback(g_last + 1, rows1, sem1)

    return body


def kernel(token_ids, embed_table):
    b0, s = token_ids.shape
    v, d = embed_table.shape
    b_flat = b0 * s
    assert b_flat % (NW * GROUP) == 0
    n_groups = b_flat // (NW * GROUP)
    table2 = embed_table.reshape(v // 2, 2 * d)
    idx_hi = (token_ids.reshape(NW, n_groups * K, CHUNK) >> 1).astype(jnp.int32)
    out2 = _build(n_groups, v // 2)(table2, idx_hi)
    return out2.reshape(b0, s, d)


# speedup vs baseline: 1.3710x; 1.3710x over previous
import functools

import jax
import jax.numpy as jnp
from jax import lax
from jax.experimental import pallas as pl
from jax.experimental.pallas import tpu as pltpu
from jax.experimental.pallas import tpu_sc as plsc

D = 64
NC = 2
NS = 16
NW = NC * NS
CHUNK = 128
K = 2
GROUP = CHUNK * K


@functools.lru_cache(maxsize=None)
def _build(n_groups):
    mesh = plsc.VectorSubcoreMesh(core_axis_name="c", subcore_axis_name="s")
    n_chunks = n_groups * K
    rows_per_w = n_groups * GROUP
    out_rows = NW * rows_per_w

    @functools.partial(
        pl.kernel,
        mesh=mesh,
        out_type=jax.ShapeDtypeStruct((out_rows, 2 * D), jnp.float32),
        scratch_types=[
            pltpu.VMEM((n_chunks, CHUNK), jnp.int32),
            pltpu.VMEM((GROUP, 2 * D), jnp.float32),
            pltpu.VMEM((GROUP, 2 * D), jnp.float32),
            pltpu.SemaphoreType.DMA,
            pltpu.SemaphoreType.DMA,
        ],
    )
    def body(table_hbm, idx_hbm, out_hbm, idx_v, rows0, rows1, sem0, sem1):
        wid = lax.axis_index("s") * NC + lax.axis_index("c")
        pltpu.sync_copy(idx_hbm.at[wid], idx_v)
        base = wid * rows_per_w

        def fire(g, buf, sem):
            for k in range(K):
                pltpu.async_copy(
                    table_hbm.at[idx_v.at[g * K + k]],
                    buf.at[pl.ds(k * CHUNK, CHUNK)],
                    sem,
                )

        def drain_writeback(g, buf, sem):
            pltpu.make_async_copy(table_hbm.at[pl.ds(0, GROUP)], buf, sem).wait()
            pltpu.sync_copy(buf, out_hbm.at[pl.ds(base + g * GROUP, GROUP)])

        fire(0, rows0, sem0)

        def step(go, carry):
            g = 2 * go
            fire(g + 1, rows1, sem1)
            drain_writeback(g, rows0, sem0)
            fire(g + 2, rows0, sem0)
            drain_writeback(g + 1, rows1, sem1)
            return carry

        lax.fori_loop(0, n_groups // 2 - 1, step, 0)

        g_last = n_groups - 2
        fire(g_last + 1, rows1, sem1)
        drain_writeback(g_last, rows0, sem0)
        drain_writeback(g_last + 1, rows1, sem1)

    return body


def kernel(token_ids, embed_table):
    b0, s = token_ids.shape
    v, d = embed_table.shape
    b_flat = b0 * s
    assert d == D and b_flat % (NW * GROUP) == 0
    n_groups = b_flat // (NW * GROUP)
    table_pad = jnp.pad(embed_table, ((0, 0), (0, D)))
    idx = token_ids.reshape(NW, n_groups * K, CHUNK).astype(jnp.int32)
    out2 = _build(n_groups)(table_pad, idx)
    return out2[:, :D].reshape(b0, s, d)
